# baseline (speedup 1.0000x reference)
import functools

import jax
import jax.numpy as jnp
from jax import lax
from jax.experimental import pallas as pl
from jax.experimental.pallas import tpu as pltpu
from jax.experimental.pallas import tpu_sc as plsc

_B = 16384
_HIST = 50
_D = 64
_NC = 2
_NS = 16
_NW = _NC * _NS
_BAGS_PER_W = _B // _NW
_CPB = 2
_IPC = 104
_NCHUNK = _BAGS_PER_W // _CPB
_NREG = _D // 16


def _sc_detile_idx(xt):
    mesh = plsc.VectorSubcoreMesh(
        core_axis_name="c", subcore_axis_name="s",
        num_cores=_NC, num_subcores=_NS,
    )

    @functools.partial(
        pl.kernel,
        out_type=jax.ShapeDtypeStruct((_HIST * _B,), jnp.int32),
        mesh=mesh,
        compiler_params=pltpu.CompilerParams(use_tc_tiling_on_sc=True),
        scratch_types=[pltpu.VMEM((_B,), jnp.int32)],
    )
    def k(xt_hbm, out_hbm, row_v):
        wid = lax.axis_index("s") * _NC + lax.axis_index("c")
        for l in range(_HIST):
            @pl.when(wid == l % _NW)
            def _row():
                pltpu.sync_copy(xt_hbm.at[l], row_v)
                pltpu.sync_copy(row_v, out_hbm.at[pl.ds(l * _B, _B)])

    return k(xt)


def _sc_embedding_sum(xl, table, emb_bias):
    mesh = plsc.VectorSubcoreMesh(
        core_axis_name="c", subcore_axis_name="s",
        num_cores=_NC, num_subcores=_NS,
    )

    @functools.partial(
        pl.kernel,
        out_type=jax.ShapeDtypeStruct((_B, _D), jnp.float32),
        mesh=mesh,
        compiler_params=pltpu.CompilerParams(
            use_tc_tiling_on_sc=False, needs_layout_passes=False),
        scratch_types=[
            pltpu.VMEM((_HIST, _BAGS_PER_W), jnp.int32),
            pltpu.VMEM((_IPC,), jnp.int32),
            pltpu.VMEM((_IPC,), jnp.int32),
            pltpu.VMEM((_IPC, _D), jnp.float32),
            pltpu.VMEM((_IPC, _D), jnp.float32),
            pltpu.VMEM((_BAGS_PER_W, _D), jnp.float32),
            pltpu.VMEM((_D,), jnp.float32),
            pltpu.SemaphoreType.DMA,
            pltpu.SemaphoreType.DMA,
        ],
    )
    def k(xl_hbm, tab_hbm, bias_hbm, out_hbm,
          idx_v, cl0, cl1, rows0, rows1, out_v, bias_v, sem0, sem1):
        wid = lax.axis_index("s") * _NC + lax.axis_index("c")
        base = wid * _BAGS_PER_W
        pltpu.sync_copy(xl_hbm.at[:, pl.ds(base, _BAGS_PER_W)], idx_v)
        pltpu.sync_copy(bias_hbm, bias_v)
        bias_regs = [bias_v[pl.ds(16 * g, 16)] for g in range(_NREG)]

        lane = lax.iota(jnp.int32, 16)
        zeros16 = jnp.zeros((16,), jnp.int32)
        _OFFS = (0, 16, 32, 34)

        def build(j, cl):
            for s in range(_CPB):
                b = _CPB * j + s
                bvec = jnp.full((16,), 0, jnp.int32) + b
                for off in _OFFS:
                    vals = plsc.load_gather(idx_v, [lane + off, bvec])
                    cl[pl.ds(_HIST * s + off, 16)] = vals

        cl0[pl.ds(88, 16)] = zeros16
        cl1[pl.ds(88, 16)] = zeros16

        def start(cl, rows, sem):
            pltpu.async_copy(tab_hbm.at[cl], rows, sem)

        def wait(cl, rows, sem):
            pltpu.make_async_copy(tab_hbm.at[cl], rows, sem).wait()

        def reduce_chunk(j, rows):
            for bag in range(_CPB):
                accs = list(bias_regs)
                for l in range(_HIST):
                    r = bag * _HIST + l
                    accs = [accs[g] + rows[r, pl.ds(16 * g, 16)]
                            for g in range(_NREG)]
                ob = j * _CPB + bag
                for g in range(_NREG):
                    out_v[ob, pl.ds(16 * g, 16)] = accs[g]

        build(0, cl0)
        start(cl0, rows0, sem0)

        def step(i, carry):
            j = 2 * i
            build(j + 1, cl1)
            start(cl1, rows1, sem1)
            wait(cl0, rows0, sem0)
            reduce_chunk(j, rows0)

            @pl.when(j + 2 < _NCHUNK)
            def _prefetch():
                build(j + 2, cl0)
                start(cl0, rows0, sem0)

            wait(cl1, rows1, sem1)
            reduce_chunk(j + 1, rows1)
            return carry

        lax.fori_loop(0, _NCHUNK // 2, step, 0)
        pltpu.sync_copy(out_v, out_hbm.at[pl.ds(base, _BAGS_PER_W)])

    return k(xl, table, emb_bias)


def kernel(x, table, emb_bias):
    xt = x.astype(jnp.int32).T
    xflat = _sc_detile_idx(xt)
    xl = xflat.reshape(_HIST, _B)
    return _sc_embedding_sum(xl, table, emb_bias)
